# baseline (speedup 1.0000x reference)
import functools

import jax
import jax.numpy as jnp
import numpy as np
from jax.experimental import pallas as pl

NB_CODE = 8192
C = 64
B = 16
J = 16
L = 16
SCALES = [(1, 1), (2, 2), (3, 3), (4, 4), (5, 5), (6, 6), (8, 8), (10, 10), (13, 13), (16, 16)]
SN = len(SCALES)
N_PHI = 4
RESI = 0.5
BETA = 0.25
TICKS = np.linspace(1.0 / 3.0 / N_PHI, 1.0 - 1.0 / 3.0 / N_PHI, N_PHI)

N_ROWS = [B * sn * tn for (sn, tn) in SCALES]
ROW_BLK = {16: 16, 64: 64, 144: 144, 256: 256, 400: 400,
           576: 288, 1024: 512, 1600: 400, 2704: 208, 4096: 512}


def _area_matrix(n_out, n_in):
    M = np.zeros((n_out, n_in), np.float32)
    for i in range(n_out):
        s = (i * n_in) // n_out
        e = -((-(i + 1) * n_in) // n_out)
        M[i, s:e] = 1.0 / (e - s)
    return M


def _cubic(x, a=-0.75):
    x = abs(float(x))
    if x <= 1.0:
        return (a + 2) * x ** 3 - (a + 3) * x ** 2 + 1.0
    if x < 2.0:
        return a * x ** 3 - 5 * a * x ** 2 + 8 * a * x - 4 * a
    return 0.0


def _bicubic_matrix(n_out, n_in):
    M = np.zeros((n_out, n_in), np.float32)
    scale = n_in / n_out
    for i in range(n_out):
        src = (i + 0.5) * scale - 0.5
        base = int(np.floor(src))
        for m in range(-1, 3):
            w = _cubic(src - (base + m))
            idx = min(max(base + m, 0), n_in - 1)
            M[i, idx] += w
    return M


def _phi_index(si):
    return int(np.argmin(np.abs(TICKS - si / (SN - 1))))


def _eye(n):
    return np.eye(n, dtype=np.float32)


_DOWN1 = [np.kron(_area_matrix(sn, J), _eye(L)).astype(np.float32)
          for (sn, tn) in SCALES[:-1]]
_DOWN2 = [np.kron(_eye(sn), _area_matrix(tn, L)).astype(np.float32)
          for (sn, tn) in SCALES[:-1]]
_UP1 = [np.kron(_bicubic_matrix(J, sn), _eye(tn)).astype(np.float32)
        for (sn, tn) in SCALES[:-1]]
_UP2 = [np.kron(_eye(J), _bicubic_matrix(L, tn)).astype(np.float32)
        for (sn, tn) in SCALES[:-1]]
_PHI_IDX = [_phi_index(si) for si in range(SN)]


def _prec(k):
    return jax.lax.Precision.HIGHEST if k > 128 else None


def _downsample(d1, d2, x):
    p1 = _prec(d1.shape[1])
    p2 = _prec(d2.shape[1])
    return jnp.concatenate(
        [jnp.dot(d2, jnp.dot(d1, x[b * J * L:(b + 1) * J * L, :],
                             preferred_element_type=jnp.float32, precision=p1),
                 preferred_element_type=jnp.float32, precision=p2) for b in range(B)], axis=0)


def _search_body(rows_ref, r2_ref, embT_ref, esq_ref, emb_ref,
                 h_ref, counts_ref, *, blk):
    r = rows_ref[...]
    score = jnp.dot(r, embT_ref[...], preferred_element_type=jnp.float32)
    d = r2_ref[...] + esq_ref[...] - 2.0 * score
    m = jnp.min(d, axis=1, keepdims=True)
    lane = jax.lax.broadcasted_iota(jnp.int32, (blk, NB_CODE), 1)
    idx = jnp.min(jnp.where(d <= m, lane, NB_CODE), axis=1, keepdims=True)
    oh = (lane == idx).astype(jnp.float32)
    h_ref[...] = jnp.dot(oh, emb_ref[...], preferred_element_type=jnp.float32,
                         precision=jax.lax.Precision.HIGHEST)
    csum = jnp.sum(oh, axis=0, keepdims=True)

    @pl.when(pl.program_id(0) == 0)
    def _init():
        counts_ref[...] = csum

    @pl.when(pl.program_id(0) != 0)
    def _acc():
        counts_ref[...] = counts_ref[...] + csum


def _make_search(n_rows):
    blk = ROW_BLK[n_rows]
    nb = n_rows // blk
    return pl.pallas_call(
        functools.partial(_search_body, blk=blk),
        grid=(nb,),
        in_specs=[
            pl.BlockSpec((blk, C), lambda i: (i, 0)),
            pl.BlockSpec((blk, 1), lambda i: (i, 0)),
            pl.BlockSpec((C, NB_CODE), lambda i: (0, 0)),
            pl.BlockSpec((1, NB_CODE), lambda i: (0, 0)),
            pl.BlockSpec((NB_CODE, C), lambda i: (0, 0)),
        ],
        out_specs=[
            pl.BlockSpec((blk, C), lambda i: (i, 0)),
            pl.BlockSpec((1, NB_CODE), lambda i: (0, 0)),
        ],
        out_shape=[
            jax.ShapeDtypeStruct((n_rows, C), jnp.float32),
            jax.ShapeDtypeStruct((1, NB_CODE), jnp.float32),
        ],
    )


def _update_body(si, h_ref, w_ref, b_ref, f_ref, frest_ref, fhat_ref,
                 um_refs, dm_refs,
                 frest_out, fhat_out, loss_ref, rows_ref):
    st = SCALES[si][0] * SCALES[si][1]
    h_rows = h_ref[...]
    if si == 0:
        up = jnp.concatenate(
            [jnp.broadcast_to(h_rows[b:b + 1, :], (J * L, C)) for b in range(B)], axis=0)
    elif um_refs is not None:
        u1 = um_refs[0][...]
        u2 = um_refs[1][...]
        p1 = _prec(u1.shape[1])
        p2 = _prec(u2.shape[1])
        up = jnp.concatenate(
            [jnp.dot(u2, jnp.dot(u1, h_rows[b * st:(b + 1) * st, :],
                                 preferred_element_type=jnp.float32, precision=p1),
                     preferred_element_type=jnp.float32, precision=p2) for b in range(B)], axis=0)
    else:
        up = h_rows

    x4 = up.reshape(B, J, L, C)
    zj = jnp.zeros((B, 1, L, C), jnp.float32)
    zl = jnp.zeros((B, J + 2, 1, C), jnp.float32)
    xp = jnp.concatenate([zj, x4, zj], axis=1)
    xp = jnp.concatenate([zl, xp, zl], axis=2)
    y = jnp.zeros((B * J * L, C), jnp.float32)
    for di in range(3):
        for dj in range(3):
            tap = xp[:, di:di + J, dj:dj + L, :].reshape(B * J * L, C)
            y = y + jnp.dot(tap, w_ref[di, dj],
                            preferred_element_type=jnp.float32)
    y = y + b_ref[0][None, :]
    hb = up * (1.0 - RESI) + y * RESI

    fv = f_ref[...]
    f_hat = fhat_ref[...] + hb
    f_rest = frest_ref[...] - hb
    diff = f_hat - fv
    m = jnp.mean(diff * diff)
    loss = m * BETA + m

    frest_out[...] = f_rest
    if si == SN - 1:
        fhat_out[...] = (f_hat - fv) + fv
    else:
        fhat_out[...] = f_hat
    loss_ref[...] = jnp.reshape(loss, (1, 1))
    if rows_ref is not None:
        rows_ref[...] = _downsample(dm_refs[0][...], dm_refs[1][...], f_rest)


def _first_rows_body(f_ref, d1_ref, d2_ref, rows_ref):
    rows_ref[...] = _downsample(d1_ref[...], d2_ref[...], f_ref[...])


def _final_body(*refs):
    counts_refs = refs[:SN]
    loss_refs = refs[SN:2 * SN]
    loss_out, perp_out = refs[2 * SN:]
    counts = counts_refs[0][...]
    for r in counts_refs[1:]:
        counts = counts + r[...]
    loss = loss_refs[0][...]
    for r in loss_refs[1:]:
        loss = loss + r[...]
    total = jnp.sum(counts)
    probs = counts / total
    perp = jnp.exp(-jnp.sum(probs * jnp.log(probs + 1e-10)))
    loss_out[...] = loss / SN
    perp_out[...] = jnp.reshape(perp, (1, 1))


@jax.jit
def kernel(f_bcjl, emb, conv_w, conv_b):
    f0 = f_bcjl.transpose(0, 2, 3, 1).reshape(B * J * L, C)
    embT = emb.T
    esq = jnp.sum(emb * emb, axis=1)[None, :]
    W = conv_w.transpose(0, 3, 4, 2, 1)

    rows = pl.pallas_call(
        _first_rows_body,
        out_shape=jax.ShapeDtypeStruct((N_ROWS[0], C), jnp.float32),
    )(f0, jnp.asarray(_DOWN1[0]), jnp.asarray(_DOWN2[0]))

    f_rest = f0
    f_hat = jnp.zeros_like(f0)
    counts_all = []
    loss_all = []
    for si in range(SN):
        n_rows = N_ROWS[si]
        r2 = jnp.sum(rows * rows, axis=1, keepdims=True)
        h_rows, counts = _make_search(n_rows)(rows, r2, embT, esq, emb)
        counts_all.append(counts)

        last = si == SN - 1
        ums = None if last else (jnp.asarray(_UP1[si]), jnp.asarray(_UP2[si]))
        dms = (None if (last or si + 1 == SN - 1)
               else (jnp.asarray(_DOWN1[si + 1]), jnp.asarray(_DOWN2[si + 1])))
        n_next = 0 if last else N_ROWS[si + 1]
        out_shape = [
            jax.ShapeDtypeStruct((B * J * L, C), jnp.float32),
            jax.ShapeDtypeStruct((B * J * L, C), jnp.float32),
            jax.ShapeDtypeStruct((1, 1), jnp.float32),
        ]
        if dms is not None:
            out_shape.append(jax.ShapeDtypeStruct((n_next, C), jnp.float32))

        def body(h_ref, w_ref, b_ref, f_ref, frest_ref, fhat_ref, *rest,
                 si=si, has_um=ums is not None, has_dm=dms is not None):
            k = 0
            um_refs = (rest[k], rest[k + 1]) if has_um else None
            k += 2 * has_um
            dm_refs = (rest[k], rest[k + 1]) if has_dm else None
            k += 2 * has_dm
            frest_out, fhat_out, loss_ref = rest[k:k + 3]
            rows_ref = rest[k + 3] if has_dm else None
            _update_body(si, h_ref, w_ref, b_ref, f_ref, frest_ref, fhat_ref,
                         um_refs, dm_refs, frest_out, fhat_out, loss_ref, rows_ref)

        pi = _PHI_IDX[si]
        ins = [h_rows, W[pi], conv_b[pi][None, :], f0, f_rest, f_hat]
        if ums is not None:
            ins.extend(ums)
        if dms is not None:
            ins.extend(dms)
        outs = pl.pallas_call(body, out_shape=out_shape)(*ins)
        f_rest, f_hat, loss_si = outs[:3]
        loss_all.append(loss_si)
        if dms is not None:
            rows = outs[3]
        else:
            rows = f_rest

    loss, perp = pl.pallas_call(
        _final_body,
        out_shape=[jax.ShapeDtypeStruct((1, 1), jnp.float32),
                   jax.ShapeDtypeStruct((1, 1), jnp.float32)],
    )(*counts_all, *loss_all)

    f_hat_out = f_hat.reshape(B, J, L, C).transpose(0, 3, 1, 2)
    return f_hat_out, loss[0, 0], perp[0, 0]
